# baseline (speedup 1.0000x reference)
import functools

import jax
import jax.numpy as jnp
from jax.experimental import pallas as pl
from jax.experimental.pallas import tpu as pltpu


def _fused_kernel(xp_ref, w1_ref, b1_ref, w2_ref, b2_ref, w3_ref, b3_ref,
                  fw1_ref, fb1_ref, fw2_ref, fb2_ref, o_ref,
                  sa, sb, w1s, w2s, w3s, fwbuf, zacc, sem, *, n, B, Lp):
    f32 = jnp.float32
    step = pl.program_id(0)

    @pl.when(step == 0)
    def _conv():

        w1s[...] = jnp.zeros_like(w1s)
        for p in range(4):
            for k in range(3):
                w1s[2 * p + k, 32 * p:32 * p + 32] = w1_ref[k, :]
                w1s[2 * p + 1 + k, 128 + 32 * p:160 + 32 * p] = w1_ref[k, :]
        w2s[...] = jnp.zeros_like(w2s)
        for g in range(6):
            for p2 in range(4):
                k = g - p2
                if 0 <= k <= 2:
                    w2s[32 * g:32 * g + 32, 64 * p2:64 * p2 + 64] = (
                        w2_ref[32 * k:32 * k + 32, :])
        w3s[...] = jnp.zeros_like(w3s)
        for g in range(4):
            for p in range(2):
                k = g - p
                if 0 <= k <= 2:
                    w3s[64 * g:64 * g + 64, 128 * p:128 * p + 128] = (
                        w3_ref[64 * k:64 * k + 64, :])

        b1t = jnp.concatenate([b1_ref[...]] * 4, axis=1)
        b2t = jnp.concatenate([b2_ref[...]] * 2, axis=1)

        ch = 2048
        nc = n // ch
        zb = jnp.zeros((B, 1), f32)
        sa[0:B, :] = jnp.zeros((B, 128), f32)
        sa[n + B:n + 2 * B, :] = jnp.zeros((B, 128), f32)
        sb[0:B, :] = jnp.zeros((B, 128), f32)
        sb[n + B:n + 2 * B, :] = jnp.zeros((B, 128), f32)

        for c in range(nc):
            r = c * ch
            xv = xp_ref[r:r + ch, :]
            if c == 0:
                pc = jnp.concatenate([zb, xp_ref[0:ch - B, 7:8]], axis=0)
            else:
                pc = xp_ref[r - B:r + ch - B, 7:8]
            if c == nc - 1:
                nx = jnp.concatenate([xp_ref[r + B:n, 0:1], zb], axis=0)
            else:
                nx = xp_ref[r + B:r + ch + B, 0:1]
            i1 = jnp.concatenate([pc, xv, nx], axis=1)
            o1 = jnp.dot(i1, w1s[0:10, :], preferred_element_type=f32)
            sa[B + r:B + r + ch, :] = jnp.maximum(
                jnp.maximum(o1[:, 0:128], o1[:, 128:256]) + b1t, 0.0)

        for c in range(nc):
            r = c * ch
            h1c = sa[B + r:B + r + ch, :]
            prev_hi = sa[r:r + ch, 96:128]
            next_lo = sa[2 * B + r:2 * B + r + ch, 0:32]
            i2 = jnp.concatenate([prev_hi, h1c, next_lo], axis=1)
            o2 = jnp.dot(i2, w2s[...], preferred_element_type=f32)
            pe = jnp.maximum(o2[:, 0:64], o2[:, 64:128])
            po = jnp.maximum(o2[:, 128:192], o2[:, 192:256])
            sb[B + r:B + r + ch, :] = jnp.maximum(
                jnp.concatenate([pe, po], axis=1) + b2t, 0.0)

        for c in range(nc):
            r = c * ch
            h2c = sb[B + r:B + r + ch, :]
            prev_hi = sb[r:r + ch, 64:128]
            next_lo = sb[2 * B + r:2 * B + r + ch, 0:64]
            i3 = jnp.concatenate([prev_hi, h2c, next_lo], axis=1)
            o3 = jnp.dot(i3, w3s[...], preferred_element_type=f32)
            sa[r:r + ch, :] = jnp.maximum(
                jnp.maximum(o3[:, 0:128], o3[:, 128:256]) + b3_ref[...], 0.0)

    @pl.when(step == 1)
    def _fc():
        pltpu.make_async_copy(fw1_ref, fwbuf, sem).start()
        pltpu.make_async_copy(fw1_ref, fwbuf, sem).wait()
        zacc[...] = jnp.zeros_like(zacc)
        for l2 in range(Lp // 2):
            hl = sa[2 * l2 * B:(2 * l2 + 2) * B, :]
            lhs = jnp.concatenate([hl[0:B, :], hl[B:2 * B, :]],
                                  axis=1)
            zacc[...] += jnp.dot(lhs, fwbuf[256 * l2:256 * (l2 + 1), :],
                                 preferred_element_type=f32)
        z = jnp.maximum(zacc[...] + fb1_ref[...], 0.0)
        out = jnp.dot(z, fw2_ref[...], preferred_element_type=f32)
        o_ref[...] = out + fb2_ref[...]


def kernel(x, edges, w1k, b1r, w2k, b2r, w3k, b3r, fw1k, fb1r, fw2k, fb2r):
    B, c0, L = x.shape
    Lp = L // 8
    n = Lp * B
    ncp = fw2k.shape[1]
    H1 = fw1k.shape[1]

    xp = x[:, 0, :].astype(jnp.float32).reshape(B, Lp, 8)
    xp = xp.transpose(1, 0, 2).reshape(n, 8)

    const = lambda i: (0, 0)
    out = pl.pallas_call(
        functools.partial(_fused_kernel, n=n, B=B, Lp=Lp),
        out_shape=jax.ShapeDtypeStruct((B, ncp), jnp.float32),
        grid=(2,),
        in_specs=[
            pl.BlockSpec((n, 8), const),
            pl.BlockSpec(w1k.shape, const),
            pl.BlockSpec(b1r.shape, const),
            pl.BlockSpec(w2k.shape, const),
            pl.BlockSpec(b2r.shape, const),
            pl.BlockSpec(w3k.shape, const),
            pl.BlockSpec(b3r.shape, const),
            pl.BlockSpec(memory_space=pl.ANY),
            pl.BlockSpec(fb1r.shape, const),
            pl.BlockSpec(fw2k.shape, const),
            pl.BlockSpec(fb2r.shape, const),
        ],
        out_specs=pl.BlockSpec((B, ncp), const),
        scratch_shapes=[
            pltpu.VMEM((n + 2 * B, 128), jnp.float32),
            pltpu.VMEM((n + 2 * B, 128), jnp.float32),
            pltpu.VMEM((16, 256), jnp.float32),
            pltpu.VMEM((192, 256), jnp.float32),
            pltpu.VMEM((256, 256), jnp.float32),
            pltpu.VMEM(fw1k.shape, jnp.float32),
            pltpu.VMEM((B, H1), jnp.float32),
            pltpu.SemaphoreType.DMA,
        ],
        compiler_params=pltpu.CompilerParams(
            dimension_semantics=("arbitrary",),
            vmem_limit_bytes=52 * 1024 * 1024,
        ),
    )(xp, w1k, b1r, w2k, b2r, w3k, b3r, fw1k, fb1r, fw2k, fb2r)

    return out


# speedup vs baseline: 1.0695x; 1.0695x over previous
import functools

import jax
import jax.numpy as jnp
from jax.experimental import pallas as pl
from jax.experimental.pallas import tpu as pltpu


def _fused_kernel(xp_ref, w1_ref, b1_ref, w2_ref, b2_ref, w3_ref, b3_ref,
                  fw1_ref, fb1_ref, fw2_ref, fb2_ref, o_ref,
                  sa, sb, w1s, w2s, w3s, fwbuf, zacc, sem, *, n, B, Lp):
    f32 = jnp.float32
    step = pl.program_id(0)

    @pl.when(step == 0)
    def _conv():
        pltpu.make_async_copy(fw1_ref, fwbuf, sem).start()

        w1s[...] = jnp.zeros_like(w1s)
        for p in range(4):
            for k in range(3):
                w1s[2 * p + k, 32 * p:32 * p + 32] = w1_ref[k, :]
                w1s[2 * p + 1 + k, 128 + 32 * p:160 + 32 * p] = w1_ref[k, :]
        w2s[...] = jnp.zeros_like(w2s)
        for g in range(6):
            for p2 in range(4):
                k = g - p2
                if 0 <= k <= 2:
                    w2s[32 * g:32 * g + 32, 64 * p2:64 * p2 + 64] = (
                        w2_ref[32 * k:32 * k + 32, :].astype(jnp.bfloat16))
        w3s[...] = jnp.zeros_like(w3s)
        for g in range(4):
            for p in range(2):
                k = g - p
                if 0 <= k <= 2:
                    w3s[64 * g:64 * g + 64, 128 * p:128 * p + 128] = (
                        w3_ref[64 * k:64 * k + 64, :].astype(jnp.bfloat16))

        b1t = jnp.concatenate([b1_ref[...]] * 4, axis=1)
        b2t = jnp.concatenate([b2_ref[...]] * 2, axis=1)

        ch = 2048
        nc = n // ch
        zb = jnp.zeros((B, 1), f32)
        zs = jnp.zeros((B, 128), jnp.bfloat16)
        sa[0:B, :] = zs
        sa[n + B:n + 2 * B, :] = zs
        sb[0:B, :] = zs
        sb[n + B:n + 2 * B, :] = zs

        for c in range(nc):
            r = c * ch
            xv = xp_ref[r:r + ch, :]
            if c == 0:
                pc = jnp.concatenate([zb, xp_ref[0:ch - B, 7:8]], axis=0)
            else:
                pc = xp_ref[r - B:r + ch - B, 7:8]
            if c == nc - 1:
                nx = jnp.concatenate([xp_ref[r + B:n, 0:1], zb], axis=0)
            else:
                nx = xp_ref[r + B:r + ch + B, 0:1]
            i1 = jnp.concatenate([pc, xv, nx], axis=1)
            o1 = jnp.dot(i1, w1s[0:10, :], preferred_element_type=f32)
            sa[B + r:B + r + ch, :] = jnp.maximum(
                jnp.maximum(o1[:, 0:128], o1[:, 128:256]) + b1t,
                0.0).astype(jnp.bfloat16)

        for c in range(nc):
            r = c * ch
            h1c = sa[B + r:B + r + ch, :]
            prev_hi = sa[r:r + ch, 96:128]
            next_lo = sa[2 * B + r:2 * B + r + ch, 0:32]
            i2 = jnp.concatenate([prev_hi, h1c, next_lo], axis=1)
            o2 = jnp.dot(i2, w2s[...], preferred_element_type=f32)
            pe = jnp.maximum(o2[:, 0:64], o2[:, 64:128])
            po = jnp.maximum(o2[:, 128:192], o2[:, 192:256])
            sb[B + r:B + r + ch, :] = jnp.maximum(
                jnp.concatenate([pe, po], axis=1) + b2t,
                0.0).astype(jnp.bfloat16)

        for c in range(nc):
            r = c * ch
            h2c = sb[B + r:B + r + ch, :]
            prev_hi = sb[r:r + ch, 64:128]
            next_lo = sb[2 * B + r:2 * B + r + ch, 0:64]
            i3 = jnp.concatenate([prev_hi, h2c, next_lo], axis=1)
            o3 = jnp.dot(i3, w3s[...], preferred_element_type=f32)
            sa[r:r + ch, :] = jnp.maximum(
                jnp.maximum(o3[:, 0:128], o3[:, 128:256]) + b3_ref[...],
                0.0).astype(jnp.bfloat16)

    @pl.when(step == 1)
    def _fc():
        pltpu.make_async_copy(fw1_ref, fwbuf, sem).wait()
        zacc[...] = jnp.zeros_like(zacc)
        for l2 in range(Lp // 2):
            hl = sa[2 * l2 * B:(2 * l2 + 2) * B, :]
            lhs = jnp.concatenate([hl[0:B, :], hl[B:2 * B, :]],
                                  axis=1).astype(f32)
            zacc[...] += jnp.dot(lhs, fwbuf[256 * l2:256 * (l2 + 1), :],
                                 preferred_element_type=f32)
        z = jnp.maximum(zacc[...] + fb1_ref[...], 0.0)
        out = jnp.dot(z, fw2_ref[...], preferred_element_type=f32)
        o_ref[...] = out + fb2_ref[...]


def kernel(x, edges, w1k, b1r, w2k, b2r, w3k, b3r, fw1k, fb1r, fw2k, fb2r):
    B, c0, L = x.shape
    Lp = L // 8
    n = Lp * B
    ncp = fw2k.shape[1]
    H1 = fw1k.shape[1]

    xp = x[:, 0, :].astype(jnp.float32).reshape(B, Lp, 8)
    xp = xp.transpose(1, 0, 2).reshape(n, 8)

    const = lambda i: (0, 0)
    out = pl.pallas_call(
        functools.partial(_fused_kernel, n=n, B=B, Lp=Lp),
        out_shape=jax.ShapeDtypeStruct((B, ncp), jnp.float32),
        grid=(2,),
        in_specs=[
            pl.BlockSpec((n, 8), const),
            pl.BlockSpec(w1k.shape, const),
            pl.BlockSpec(b1r.shape, const),
            pl.BlockSpec(w2k.shape, const),
            pl.BlockSpec(b2r.shape, const),
            pl.BlockSpec(w3k.shape, const),
            pl.BlockSpec(b3r.shape, const),
            pl.BlockSpec(memory_space=pl.ANY),
            pl.BlockSpec(fb1r.shape, const),
            pl.BlockSpec(fw2k.shape, const),
            pl.BlockSpec(fb2r.shape, const),
        ],
        out_specs=pl.BlockSpec((B, ncp), const),
        scratch_shapes=[
            pltpu.VMEM((n + 2 * B, 128), jnp.bfloat16),
            pltpu.VMEM((n + 2 * B, 128), jnp.bfloat16),
            pltpu.VMEM((16, 256), jnp.float32),
            pltpu.VMEM((192, 256), jnp.bfloat16),
            pltpu.VMEM((256, 256), jnp.bfloat16),
            pltpu.VMEM(fw1k.shape, jnp.float32),
            pltpu.VMEM((B, H1), jnp.float32),
            pltpu.SemaphoreType.DMA,
        ],
        compiler_params=pltpu.CompilerParams(
            dimension_semantics=("arbitrary",),
            vmem_limit_bytes=52 * 1024 * 1024,
        ),
    )(xp, w1k, b1r, w2k, b2r, w3k, b3r, fw1k, fb1r, fw2k, fb2r)

    return out


# speedup vs baseline: 1.1373x; 1.0633x over previous
import functools

import jax
import jax.numpy as jnp
from jax.experimental import pallas as pl
from jax.experimental.pallas import tpu as pltpu


def _fused_kernel(xp_ref, w1_ref, b1_ref, w2_ref, b2_ref, w3_ref, b3_ref,
                  fw1_ref, fb1_ref, fw2_ref, fb2_ref, o_ref,
                  sa, sb, w1s, w2s, w3s, fwbuf, zacc, sem, *, n, B, Lp):
    f32 = jnp.float32
    step = pl.program_id(0)

    @pl.when(step == 0)
    def _conv():
        pltpu.make_async_copy(fw1_ref, fwbuf, sem).start()

        w1s[...] = jnp.zeros_like(w1s)
        for p in range(4):
            for k in range(3):
                w1s[2 * p + k, 32 * p:32 * p + 32] = w1_ref[k, :]
                w1s[2 * p + 1 + k, 128 + 32 * p:160 + 32 * p] = w1_ref[k, :]
        w2s[...] = jnp.zeros_like(w2s)
        for g in range(6):
            for p2 in range(4):
                k = g - p2
                if 0 <= k <= 2:
                    w2s[32 * g:32 * g + 32, 64 * p2:64 * p2 + 64] = (
                        w2_ref[32 * k:32 * k + 32, :])
        w3s[...] = jnp.zeros_like(w3s)
        for g in range(4):
            for p in range(2):
                k = g - p
                if 0 <= k <= 2:
                    w3s[64 * g:64 * g + 64, 128 * p:128 * p + 128] = (
                        w3_ref[64 * k:64 * k + 64, :])

        b1t = jnp.concatenate([b1_ref[...]] * 4, axis=1)
        b2t = jnp.concatenate([b2_ref[...]] * 2, axis=1)

        ch = 4096
        nc = n // ch
        zb = jnp.zeros((B, 1), f32)
        sa[0:B, :] = jnp.zeros((B, 128), f32)
        sa[n + B:n + 2 * B, :] = jnp.zeros((B, 128), f32)
        sb[0:B, :] = jnp.zeros((B, 128), f32)
        sb[n + B:n + 2 * B, :] = jnp.zeros((B, 128), f32)

        for c in range(nc):
            r = c * ch
            xv = xp_ref[r:r + ch, :]
            if c == 0:
                pc = jnp.concatenate([zb, xp_ref[0:ch - B, 7:8]], axis=0)
            else:
                pc = xp_ref[r - B:r + ch - B, 7:8]
            if c == nc - 1:
                nx = jnp.concatenate([xp_ref[r + B:n, 0:1], zb], axis=0)
            else:
                nx = xp_ref[r + B:r + ch + B, 0:1]
            i1 = jnp.concatenate([pc, xv, nx], axis=1)
            o1 = jnp.dot(i1, w1s[0:10, :], preferred_element_type=f32)
            sa[B + r:B + r + ch, :] = jnp.maximum(
                jnp.maximum(o1[:, 0:128], o1[:, 128:256]) + b1t, 0.0)

        for c in range(nc):
            r = c * ch
            h1c = sa[B + r:B + r + ch, :]
            prev_hi = sa[r:r + ch, 96:128]
            next_lo = sa[2 * B + r:2 * B + r + ch, 0:32]
            i2 = jnp.concatenate([prev_hi, h1c, next_lo], axis=1)
            o2 = jnp.dot(i2, w2s[...], preferred_element_type=f32)
            pe = jnp.maximum(o2[:, 0:64], o2[:, 64:128])
            po = jnp.maximum(o2[:, 128:192], o2[:, 192:256])
            sb[B + r:B + r + ch, :] = jnp.maximum(
                jnp.concatenate([pe, po], axis=1) + b2t, 0.0)

        for c in range(nc):
            r = c * ch
            h2c = sb[B + r:B + r + ch, :]
            prev_hi = sb[r:r + ch, 64:128]
            next_lo = sb[2 * B + r:2 * B + r + ch, 0:64]
            i3 = jnp.concatenate([prev_hi, h2c, next_lo], axis=1)
            o3 = jnp.dot(i3, w3s[...], preferred_element_type=f32)
            sa[r:r + ch, :] = jnp.maximum(
                jnp.maximum(o3[:, 0:128], o3[:, 128:256]) + b3_ref[...], 0.0)

    @pl.when(step == 1)
    def _fc():
        pltpu.make_async_copy(fw1_ref, fwbuf, sem).wait()
        zacc[...] = jnp.zeros_like(zacc)
        for l2 in range(Lp // 2):
            hl = sa[2 * l2 * B:(2 * l2 + 2) * B, :]
            lhs = jnp.concatenate([hl[0:B, :], hl[B:2 * B, :]],
                                  axis=1)
            zacc[...] += jnp.dot(lhs, fwbuf[256 * l2:256 * (l2 + 1), :],
                                 preferred_element_type=f32)
        z = jnp.maximum(zacc[...] + fb1_ref[...], 0.0)
        out = jnp.dot(z, fw2_ref[...], preferred_element_type=f32)
        o_ref[...] = out + fb2_ref[...]


def kernel(x, edges, w1k, b1r, w2k, b2r, w3k, b3r, fw1k, fb1r, fw2k, fb2r):
    B, c0, L = x.shape
    Lp = L // 8
    n = Lp * B
    ncp = fw2k.shape[1]
    H1 = fw1k.shape[1]

    xp = x[:, 0, :].astype(jnp.float32).reshape(B, Lp, 8)
    xp = xp.transpose(1, 0, 2).reshape(n, 8)

    const = lambda i: (0, 0)
    out = pl.pallas_call(
        functools.partial(_fused_kernel, n=n, B=B, Lp=Lp),
        out_shape=jax.ShapeDtypeStruct((B, ncp), jnp.float32),
        grid=(2,),
        in_specs=[
            pl.BlockSpec((n, 8), const),
            pl.BlockSpec(w1k.shape, const),
            pl.BlockSpec(b1r.shape, const),
            pl.BlockSpec(w2k.shape, const),
            pl.BlockSpec(b2r.shape, const),
            pl.BlockSpec(w3k.shape, const),
            pl.BlockSpec(b3r.shape, const),
            pl.BlockSpec(memory_space=pl.ANY),
            pl.BlockSpec(fb1r.shape, const),
            pl.BlockSpec(fw2k.shape, const),
            pl.BlockSpec(fb2r.shape, const),
        ],
        out_specs=pl.BlockSpec((B, ncp), const),
        scratch_shapes=[
            pltpu.VMEM((n + 2 * B, 128), jnp.float32),
            pltpu.VMEM((n + 2 * B, 128), jnp.float32),
            pltpu.VMEM((16, 256), jnp.float32),
            pltpu.VMEM((192, 256), jnp.float32),
            pltpu.VMEM((256, 256), jnp.float32),
            pltpu.VMEM(fw1k.shape, jnp.float32),
            pltpu.VMEM((B, H1), jnp.float32),
            pltpu.SemaphoreType.DMA,
        ],
        compiler_params=pltpu.CompilerParams(
            dimension_semantics=("arbitrary",),
            vmem_limit_bytes=52 * 1024 * 1024,
        ),
    )(xp, w1k, b1r, w2k, b2r, w3k, b3r, fw1k, fb1r, fw2k, fb2r)

    return out
